# baseline (speedup 1.0000x reference)
import functools

import jax
import jax.numpy as jnp
import numpy as np
from jax import lax
from jax.experimental import pallas as pl
from jax.experimental.pallas import tpu as pltpu
from jax.experimental.pallas import tpu_sc as plsc

B, N_DOCS, L_Q, L_D, D = 1024, 50, 20, 50, 32
SEGS = B * N_DOCS
VOCAB = 1000000 + 10

NC, NS = 2, 16
NW = NC * NS

SEG_W = SEGS // NW
CH_SEGS = 32
CHUNKS = SEG_W // CH_SEGS
CH_TOK = CH_SEGS * L_D
GATHER_N = 80
N_GATHERS = CH_TOK // GATHER_N

QB_W = B // NW
Q_TOK_W = QB_W * L_Q
QGATHER_N = 80
NQ_GATHERS = Q_TOK_W // QGATHER_N

_HALF_PERM = np.concatenate([np.arange(0, D, 2), np.arange(1, D, 2)])
_X_COL_PERM = np.concatenate([_HALF_PERM, D + _HALF_PERM])

_F32 = jnp.float32


def _unpack_row(v_bf16):
    return plsc.unpack(v_bf16, format=plsc.PackFormat.INTERLEAVED)


def _sc_pool_kernel(qtok, dtok, table, x_out,
                    idx0, idx1, rows0, rows1, xbuf0, xbuf1, qpool, qidx,
                    sem0, sem1):
    wid = lax.axis_index("s") * NC + lax.axis_index("c")

    pltpu.sync_copy(qtok.at[pl.ds(wid * Q_TOK_W, Q_TOK_W)], qidx)
    qd = [
        pltpu.async_copy(
            table.at[qidx.at[pl.ds(j * QGATHER_N, QGATHER_N)]],
            rows0.at[pl.ds(j * QGATHER_N, QGATHER_N)],
            sem0,
        )
        for j in range(NQ_GATHERS)
    ]
    for d in qd:
        d.wait()

    def q_body(qb, _):
        a0 = jnp.zeros((16,), _F32)
        a1 = jnp.zeros((16,), _F32)
        for l in range(L_Q):
            r = qb * L_Q + l
            fe, fo = _unpack_row(rows0[r, 0:D])
            a0 = a0 + fe
            a1 = a1 + fo
        qpool[qb, 0:16] = a0
        qpool[qb, 16:32] = a1
        return 0

    lax.fori_loop(0, QB_W, q_body, 0, unroll=False)

    def fire(ci, idx, rows, sem):
        tok_base = wid * (SEG_W * L_D) + ci * CH_TOK
        pltpu.sync_copy(dtok.at[pl.ds(tok_base, CH_TOK)], idx)
        for j in range(N_GATHERS):
            pltpu.async_copy(
                table.at[idx.at[pl.ds(j * GATHER_N, GATHER_N)]],
                rows.at[pl.ds(j * GATHER_N, GATHER_N)],
                sem,
            )

    def drain(rows, sem):
        for j in range(N_GATHERS):
            pltpu.make_async_copy(
                table.at[pl.ds(0, GATHER_N)],
                rows.at[pl.ds(j * GATHER_N, GATHER_N)],
                sem,
            ).wait()

    def compute(ci, rows, xbuf):
        seg_base = wid * SEG_W + ci * CH_SEGS

        def seg_body(si, _):
            a0 = jnp.zeros((16,), _F32)
            a1 = jnp.zeros((16,), _F32)
            for l in range(L_D):
                r = si * L_D + l
                fe, fo = _unpack_row(rows[r, 0:D])
                a0 = a0 + fe
                a1 = a1 + fo
            qb = (seg_base + si) // N_DOCS - wid * QB_W
            xbuf[si, 0:16] = qpool[qb, 0:16]
            xbuf[si, 16:32] = qpool[qb, 16:32]
            xbuf[si, 32:48] = a0
            xbuf[si, 48:64] = a1
            return 0

        lax.fori_loop(0, CH_SEGS, seg_body, 0, unroll=False)
        pltpu.sync_copy(xbuf, x_out.at[pl.ds(seg_base, CH_SEGS)])

    fire(0, idx0, rows0, sem0)

    def pair_body(i, _):
        fire(2 * i + 1, idx1, rows1, sem1)
        drain(rows0, sem0)
        compute(2 * i, rows0, xbuf0)

        @pl.when(i < CHUNKS // 2 - 1)
        def _():
            fire(2 * i + 2, idx0, rows0, sem0)

        drain(rows1, sem1)
        compute(2 * i + 1, rows1, xbuf1)
        return 0

    lax.fori_loop(0, CHUNKS // 2, pair_body, 0, unroll=False)


@functools.lru_cache(maxsize=1)
def _make_sc_pool():
    mesh = plsc.VectorSubcoreMesh(
        core_axis_name="c", subcore_axis_name="s", num_cores=NC, num_subcores=NS
    )
    return pl.kernel(
        _sc_pool_kernel,
        out_type=jax.ShapeDtypeStruct((SEGS, 2 * D), _F32),
        mesh=mesh,
        scratch_types=[
            pltpu.VMEM((CH_TOK,), jnp.int32),
            pltpu.VMEM((CH_TOK,), jnp.int32),
            pltpu.VMEM((CH_TOK, D), jnp.bfloat16),
            pltpu.VMEM((CH_TOK, D), jnp.bfloat16),
            pltpu.VMEM((CH_SEGS, 2 * D), _F32),
            pltpu.VMEM((CH_SEGS, 2 * D), _F32),
            pltpu.VMEM((QB_W, D), _F32),
            pltpu.VMEM((Q_TOK_W,), jnp.int32),
            pltpu.SemaphoreType.DMA,
            pltpu.SemaphoreType.DMA,
        ],
        compiler_params=pltpu.CompilerParams(
            use_tc_tiling_on_sc=False, needs_layout_passes=False
        ),
    )


_CB = 8192
_CBQ = _CB // 4
_CBQ_SHIFT = _CBQ.bit_length() - 1
_RP_GRID = -(-VOCAB // _CB)
_RP_ROWS = _RP_GRID * _CB // 4


def _repack_body(in_ref, o_ref):
    xt = in_ref[...].T.astype(jnp.bfloat16)
    q = _CBQ
    o_ref[...] = jnp.concatenate(
        [xt[0:q], xt[q:2 * q], xt[2 * q:3 * q], xt[3 * q:4 * q]], axis=1
    )


def _gather_index(t):
    return (t & -_CB) + ((t & (_CBQ - 1)) << 2) + ((t >> _CBQ_SHIFT) & 3)


def _repack(tableT):
    return pl.pallas_call(
        _repack_body,
        grid=(_RP_GRID,),
        in_specs=[pl.BlockSpec((D, _CB), lambda i: (0, i))],
        out_specs=pl.BlockSpec((_CB // 4, 128), lambda i: (i, 0)),
        out_shape=jax.ShapeDtypeStruct((_RP_ROWS, 128), jnp.bfloat16),
    )(tableT)


def _mlp_body(x_ref, w1_ref, b1_ref, w2_ref, b2_ref, w3_ref, b3_ref, o_ref):
    h = jnp.maximum(
        jnp.dot(x_ref[...], w1_ref[...], preferred_element_type=_F32) + b1_ref[...], 0.0
    )
    h = jnp.maximum(
        jnp.dot(h, w2_ref[...], preferred_element_type=_F32) + b2_ref[...], 0.0
    )
    s = jnp.sum(h * w3_ref[...], axis=1, keepdims=True) + b3_ref[...]
    o_ref[...] = s


def _mlp(x, w1, b1, w2, b2, w3row, b3):
    rows_blk = SEGS // 8
    return pl.pallas_call(
        _mlp_body,
        grid=(8,),
        in_specs=[
            pl.BlockSpec((rows_blk, 2 * D), lambda i: (i, 0)),
            pl.BlockSpec((2 * D, 64), lambda i: (0, 0)),
            pl.BlockSpec((1, 64), lambda i: (0, 0)),
            pl.BlockSpec((64, 32), lambda i: (0, 0)),
            pl.BlockSpec((1, 32), lambda i: (0, 0)),
            pl.BlockSpec((1, 32), lambda i: (0, 0)),
            pl.BlockSpec((1, 1), lambda i: (0, 0)),
        ],
        out_specs=pl.BlockSpec((rows_blk, 1), lambda i: (i, 0)),
        out_shape=jax.ShapeDtypeStruct((SEGS, 1), _F32),
    )(x, w1, b1, w2, b2, w3row, b3)


def kernel(query_tokens, document_tokens, table, bn_gamma, bn_beta, bn_mean, bn_var,
           W1, b1, W2, b2, W3, b3):
    qtok = _gather_index(query_tokens.reshape(-1))
    dtok = _gather_index(document_tokens.reshape(-1))
    table_lin = _repack(table.T).reshape(_RP_ROWS * 4, D)

    x = _make_sc_pool()(qtok, dtok, table_lin)

    inv_std = lax.rsqrt(bn_var + 1e-3)
    scale = bn_gamma * inv_std
    shift = bn_beta - bn_mean * scale
    div = jnp.concatenate(
        [jnp.full((D,), 1.0 / L_Q, _F32), jnp.full((D,), 1.0 / L_D, _F32)]
    )
    w1p = (scale * div)[:, None] * W1
    b1p = (shift @ W1 + b1).reshape(1, 64)
    w1p = w1p[_X_COL_PERM, :]

    scores = _mlp(
        x, w1p, b1p, W2, b2.reshape(1, 32), W3.reshape(1, 32), b3.reshape(1, 1)
    )
    return scores.reshape(B, N_DOCS)


# speedup vs baseline: 1.0272x; 1.0272x over previous
import functools

import jax
import jax.numpy as jnp
import numpy as np
from jax import lax
from jax.experimental import pallas as pl
from jax.experimental.pallas import tpu as pltpu
from jax.experimental.pallas import tpu_sc as plsc

B, N_DOCS, L_Q, L_D, D = 1024, 50, 20, 50, 32
SEGS = B * N_DOCS
VOCAB = 1000000 + 10

NC, NS = 2, 16
NW = NC * NS

SEG_W = SEGS // NW
CH_SEGS = 32
CHUNKS = SEG_W // CH_SEGS
CH_TOK = CH_SEGS * L_D
GATHER_N = 80
N_GATHERS = CH_TOK // GATHER_N

QB_W = B // NW
Q_TOK_W = QB_W * L_Q
QGATHER_N = 80
NQ_GATHERS = Q_TOK_W // QGATHER_N

_F32 = jnp.float32


def _unpack_row(v):
    flo = plsc.bitcast(v << 16, _F32)
    fhi = plsc.bitcast(v & jnp.int32(-65536), _F32)
    return flo, fhi


def _sc_pool_kernel(qtok, dtok, table, x_out,
                    idx0, idx1, rows0, rows1, xbuf0, xbuf1, qpool, qidx,
                    sem0, sem1):
    wid = lax.axis_index("s") * NC + lax.axis_index("c")

    pltpu.sync_copy(qtok.at[pl.ds(wid * Q_TOK_W, Q_TOK_W)], qidx)
    qd = [
        pltpu.async_copy(
            table.at[qidx.at[pl.ds(j * QGATHER_N, QGATHER_N)]],
            rows0.at[pl.ds(j * QGATHER_N, QGATHER_N)],
            sem0,
        )
        for j in range(NQ_GATHERS)
    ]
    for d in qd:
        d.wait()

    def q_body(qb, _):
        a0 = jnp.zeros((16,), _F32)
        a1 = jnp.zeros((16,), _F32)
        for l in range(L_Q):
            r = qb * L_Q + l
            fe, fo = _unpack_row(rows0[r, 0:D])
            a0 = a0 + fe
            a1 = a1 + fo
        qpool[qb, 0:16] = a0
        qpool[qb, 16:32] = a1
        return 0

    lax.fori_loop(0, QB_W, q_body, 0, unroll=False)

    def fire(ci, idx, rows, sem):
        tok_base = wid * (SEG_W * L_D) + ci * CH_TOK
        pltpu.sync_copy(dtok.at[pl.ds(tok_base, CH_TOK)], idx)
        for j in range(N_GATHERS):
            pltpu.async_copy(
                table.at[idx.at[pl.ds(j * GATHER_N, GATHER_N)]],
                rows.at[pl.ds(j * GATHER_N, GATHER_N)],
                sem,
            )

    def drain(rows, sem):
        for j in range(N_GATHERS):
            pltpu.make_async_copy(
                table.at[pl.ds(0, GATHER_N)],
                rows.at[pl.ds(j * GATHER_N, GATHER_N)],
                sem,
            ).wait()

    def compute(ci, rows, xbuf):
        seg_base = wid * SEG_W + ci * CH_SEGS

        def seg_body(si, _):
            a0 = jnp.zeros((16,), _F32)
            a1 = jnp.zeros((16,), _F32)
            for l in range(L_D):
                r = si * L_D + l
                fe, fo = _unpack_row(rows[r, 0:D])
                a0 = a0 + fe
                a1 = a1 + fo
            qb = (seg_base + si) // N_DOCS - wid * QB_W
            xbuf[si, 0:16] = qpool[qb, 0:16]
            xbuf[si, 16:32] = qpool[qb, 16:32]
            xbuf[si, 32:48] = a0
            xbuf[si, 48:64] = a1
            return 0

        lax.fori_loop(0, CH_SEGS, seg_body, 0, unroll=False)
        pltpu.sync_copy(xbuf, x_out.at[pl.ds(seg_base, CH_SEGS)])

    fire(0, idx0, rows0, sem0)

    def pair_body(i, _):
        fire(2 * i + 1, idx1, rows1, sem1)
        drain(rows0, sem0)
        compute(2 * i, rows0, xbuf0)

        @pl.when(i < CHUNKS // 2 - 1)
        def _():
            fire(2 * i + 2, idx0, rows0, sem0)

        drain(rows1, sem1)
        compute(2 * i + 1, rows1, xbuf1)
        return 0

    lax.fori_loop(0, CHUNKS // 2, pair_body, 0, unroll=False)


@functools.lru_cache(maxsize=1)
def _make_sc_pool():
    mesh = plsc.VectorSubcoreMesh(
        core_axis_name="c", subcore_axis_name="s", num_cores=NC, num_subcores=NS
    )
    return pl.kernel(
        _sc_pool_kernel,
        out_type=jax.ShapeDtypeStruct((SEGS, 2 * D), _F32),
        mesh=mesh,
        scratch_types=[
            pltpu.VMEM((CH_TOK,), jnp.int32),
            pltpu.VMEM((CH_TOK,), jnp.int32),
            pltpu.VMEM((CH_TOK, 16), jnp.int32),
            pltpu.VMEM((CH_TOK, 16), jnp.int32),
            pltpu.VMEM((CH_SEGS, 2 * D), _F32),
            pltpu.VMEM((CH_SEGS, 2 * D), _F32),
            pltpu.VMEM((QB_W, D), _F32),
            pltpu.VMEM((Q_TOK_W,), jnp.int32),
            pltpu.SemaphoreType.DMA,
            pltpu.SemaphoreType.DMA,
        ],
        compiler_params=pltpu.CompilerParams(
            use_tc_tiling_on_sc=False, needs_layout_passes=False
        ),
    )


_CB = 8192
_CBQ = _CB // 8
_CBQ_SHIFT = _CBQ.bit_length() - 1
_RP_GRID = -(-VOCAB // _CB)
_RP_ROWS = _RP_GRID * _CB // 8


def _repack_body(in_ref, o_ref):
    xt = in_ref[...].T
    vi = lax.bitcast_convert_type(xt, jnp.int32) + jnp.int32(0x8000)
    lo = lax.shift_right_logical(vi[:, 0:16], 16)
    hi = vi[:, 16:32] & jnp.int32(-65536)
    pk = lo | hi
    q = _CBQ
    o_ref[...] = jnp.concatenate([pk[a * q:(a + 1) * q] for a in range(8)], axis=1)


def _gather_index(t):
    return (t & -_CB) + ((t & (_CBQ - 1)) << 3) + ((t >> _CBQ_SHIFT) & 7)


def _repack(tableT):
    return pl.pallas_call(
        _repack_body,
        grid=(_RP_GRID,),
        in_specs=[pl.BlockSpec((D, _CB), lambda i: (0, i))],
        out_specs=pl.BlockSpec((_CB // 8, 128), lambda i: (i, 0)),
        out_shape=jax.ShapeDtypeStruct((_RP_ROWS, 128), jnp.int32),
    )(tableT)


def _mlp_body(x_ref, w1_ref, b1_ref, w2_ref, b2_ref, w3_ref, b3_ref, o_ref):
    h = jnp.maximum(
        jnp.dot(x_ref[...], w1_ref[...], preferred_element_type=_F32) + b1_ref[...], 0.0
    )
    h = jnp.maximum(
        jnp.dot(h, w2_ref[...], preferred_element_type=_F32) + b2_ref[...], 0.0
    )
    s = jnp.sum(h * w3_ref[...], axis=1, keepdims=True) + b3_ref[...]
    o_ref[...] = s


def _mlp(x, w1, b1, w2, b2, w3row, b3):
    rows_blk = SEGS // 8
    return pl.pallas_call(
        _mlp_body,
        grid=(8,),
        in_specs=[
            pl.BlockSpec((rows_blk, 2 * D), lambda i: (i, 0)),
            pl.BlockSpec((2 * D, 64), lambda i: (0, 0)),
            pl.BlockSpec((1, 64), lambda i: (0, 0)),
            pl.BlockSpec((64, 32), lambda i: (0, 0)),
            pl.BlockSpec((1, 32), lambda i: (0, 0)),
            pl.BlockSpec((1, 32), lambda i: (0, 0)),
            pl.BlockSpec((1, 1), lambda i: (0, 0)),
        ],
        out_specs=pl.BlockSpec((rows_blk, 1), lambda i: (i, 0)),
        out_shape=jax.ShapeDtypeStruct((SEGS, 1), _F32),
    )(x, w1, b1, w2, b2, w3row, b3)


def kernel(query_tokens, document_tokens, table, bn_gamma, bn_beta, bn_mean, bn_var,
           W1, b1, W2, b2, W3, b3):
    qtok = _gather_index(query_tokens.reshape(-1))
    dtok = _gather_index(document_tokens.reshape(-1))
    table_lin = _repack(table.T).reshape(_RP_ROWS * 8, 16)

    x = _make_sc_pool()(qtok, dtok, table_lin)

    inv_std = lax.rsqrt(bn_var + 1e-3)
    scale = bn_gamma * inv_std
    shift = bn_beta - bn_mean * scale
    div = jnp.concatenate(
        [jnp.full((D,), 1.0 / L_Q, _F32), jnp.full((D,), 1.0 / L_D, _F32)]
    )
    w1p = (scale * div)[:, None] * W1
    b1p = (shift @ W1 + b1).reshape(1, 64)

    scores = _mlp(
        x, w1p, b1p, W2, b2.reshape(1, 32), W3.reshape(1, 32), b3.reshape(1, 1)
    )
    return scores.reshape(B, N_DOCS)


# speedup vs baseline: 1.3428x; 1.3073x over previous
import functools

import jax
import jax.numpy as jnp
import numpy as np
from jax import lax
from jax.experimental import pallas as pl
from jax.experimental.pallas import tpu as pltpu
from jax.experimental.pallas import tpu_sc as plsc

B, N_DOCS, L_Q, L_D, D = 1024, 50, 20, 50, 32
SEGS = B * N_DOCS
VOCAB = 1000000 + 10

NC, NS = 2, 16
NW = NC * NS

SEG_W = SEGS // NW
CH_SEGS = 32
CHUNKS = SEG_W // CH_SEGS
CH_TOK = CH_SEGS * L_D
GATHER_N = 80
N_GATHERS = CH_TOK // GATHER_N

QB_W = B // NW
Q_TOK_W = QB_W * L_Q
QGATHER_N = 80
NQ_GATHERS = Q_TOK_W // QGATHER_N

_F32 = jnp.float32


def _sc_pool_kernel(qtok, dtok, table, x_out,
                    idx0, idx1, rows0, rows1, xbuf0, xbuf1, qpool, qidx,
                    sem0, sem1):
    wid = lax.axis_index("s") * NC + lax.axis_index("c")

    pltpu.sync_copy(qtok.at[pl.ds(wid * Q_TOK_W, Q_TOK_W)], qidx)
    qd = [
        pltpu.async_copy(
            table.at[qidx.at[pl.ds(j * QGATHER_N, QGATHER_N)]],
            rows0.at[pl.ds(j * QGATHER_N, QGATHER_N)],
            sem0,
        )
        for j in range(NQ_GATHERS)
    ]
    for d in qd:
        d.wait()

    def q_body(qb, _):
        a0 = jnp.zeros((16,), _F32)
        a1 = jnp.zeros((16,), _F32)
        for l in range(L_Q):
            r = qb * L_Q + l
            a0 = a0 + rows0[r, 0:16]
            a1 = a1 + rows0[r, 16:32]
        qpool[qb, 0:16] = a0
        qpool[qb, 16:32] = a1
        return 0

    lax.fori_loop(0, QB_W, q_body, 0, unroll=False)

    def fire(ci, idx, rows, sem):
        tok_base = wid * (SEG_W * L_D) + ci * CH_TOK
        pltpu.sync_copy(dtok.at[pl.ds(tok_base, CH_TOK)], idx)
        for j in range(N_GATHERS):
            pltpu.async_copy(
                table.at[idx.at[pl.ds(j * GATHER_N, GATHER_N)]],
                rows.at[pl.ds(j * GATHER_N, GATHER_N)],
                sem,
            )

    def drain(rows, sem):
        for j in range(N_GATHERS):
            pltpu.make_async_copy(
                table.at[pl.ds(0, GATHER_N)],
                rows.at[pl.ds(j * GATHER_N, GATHER_N)],
                sem,
            ).wait()

    def compute(ci, rows, xbuf):
        seg_base = wid * SEG_W + ci * CH_SEGS

        def seg_body(si, _):
            a0 = jnp.zeros((16,), _F32)
            a1 = jnp.zeros((16,), _F32)
            for l in range(L_D):
                r = si * L_D + l
                a0 = a0 + rows[r, 0:16]
                a1 = a1 + rows[r, 16:32]
            qb = (seg_base + si) // N_DOCS - wid * QB_W
            xbuf[si, 0:16] = qpool[qb, 0:16]
            xbuf[si, 16:32] = qpool[qb, 16:32]
            xbuf[si, 32:48] = a0
            xbuf[si, 48:64] = a1
            return 0

        lax.fori_loop(0, CH_SEGS, seg_body, 0, unroll=False)
        pltpu.sync_copy(xbuf, x_out.at[pl.ds(seg_base, CH_SEGS)])

    fire(0, idx0, rows0, sem0)

    def pair_body(i, _):
        fire(2 * i + 1, idx1, rows1, sem1)
        drain(rows0, sem0)
        compute(2 * i, rows0, xbuf0)

        @pl.when(i < CHUNKS // 2 - 1)
        def _():
            fire(2 * i + 2, idx0, rows0, sem0)

        drain(rows1, sem1)
        compute(2 * i + 1, rows1, xbuf1)
        return 0

    lax.fori_loop(0, CHUNKS // 2, pair_body, 0, unroll=False)


@functools.lru_cache(maxsize=1)
def _make_sc_pool():
    mesh = plsc.VectorSubcoreMesh(
        core_axis_name="c", subcore_axis_name="s", num_cores=NC, num_subcores=NS
    )
    return pl.kernel(
        _sc_pool_kernel,
        out_type=jax.ShapeDtypeStruct((SEGS, 2 * D), _F32),
        mesh=mesh,
        scratch_types=[
            pltpu.VMEM((CH_TOK,), jnp.int32),
            pltpu.VMEM((CH_TOK,), jnp.int32),
            pltpu.VMEM((CH_TOK, D), _F32),
            pltpu.VMEM((CH_TOK, D), _F32),
            pltpu.VMEM((CH_SEGS, 2 * D), _F32),
            pltpu.VMEM((CH_SEGS, 2 * D), _F32),
            pltpu.VMEM((QB_W, D), _F32),
            pltpu.VMEM((Q_TOK_W,), jnp.int32),
            pltpu.SemaphoreType.DMA,
            pltpu.SemaphoreType.DMA,
        ],
        compiler_params=pltpu.CompilerParams(
            use_tc_tiling_on_sc=False, needs_layout_passes=False
        ),
    )


_CB = 8192
_CBQ = _CB // 4
_CBQ_SHIFT = _CBQ.bit_length() - 1
_RP_GRID = -(-VOCAB // _CB)
_RP_ROWS = _RP_GRID * _CB // 4


def _repack_body(in_ref, eye_ref, o_ref):
    q = _CBQ
    acc = None
    for a in range(4):
        part = lax.dot_general(
            in_ref[:, a * q:(a + 1) * q],
            eye_ref[32 * a:32 * a + 32, :],
            (((0,), (0,)), ((), ())),
            preferred_element_type=_F32,
        )
        acc = part if acc is None else acc + part
    o_ref[...] = acc


def _gather_index(t):
    return (t & -_CB) + ((t & (_CBQ - 1)) << 2) + ((t >> _CBQ_SHIFT) & 3)


def _repack(tableT):
    return pl.pallas_call(
        _repack_body,
        grid=(_RP_GRID,),
        in_specs=[
            pl.BlockSpec((D, _CB), lambda i: (0, i)),
            pl.BlockSpec((128, 128), lambda i: (0, 0)),
        ],
        out_specs=pl.BlockSpec((_CBQ, 128), lambda i: (i, 0)),
        out_shape=jax.ShapeDtypeStruct((_RP_ROWS, 128), _F32),
    )(tableT, jnp.eye(128, dtype=_F32))


def _mlp_body(x_ref, w1_ref, b1_ref, w2_ref, b2_ref, w3_ref, b3_ref, o_ref):
    h = jnp.maximum(
        jnp.dot(x_ref[...], w1_ref[...], preferred_element_type=_F32) + b1_ref[...], 0.0
    )
    h = jnp.maximum(
        jnp.dot(h, w2_ref[...], preferred_element_type=_F32) + b2_ref[...], 0.0
    )
    s = jnp.sum(h * w3_ref[...], axis=1, keepdims=True) + b3_ref[...]
    o_ref[...] = s


def _mlp(x, w1, b1, w2, b2, w3row, b3):
    rows_blk = SEGS // 8
    return pl.pallas_call(
        _mlp_body,
        grid=(8,),
        in_specs=[
            pl.BlockSpec((rows_blk, 2 * D), lambda i: (i, 0)),
            pl.BlockSpec((2 * D, 64), lambda i: (0, 0)),
            pl.BlockSpec((1, 64), lambda i: (0, 0)),
            pl.BlockSpec((64, 32), lambda i: (0, 0)),
            pl.BlockSpec((1, 32), lambda i: (0, 0)),
            pl.BlockSpec((1, 32), lambda i: (0, 0)),
            pl.BlockSpec((1, 1), lambda i: (0, 0)),
        ],
        out_specs=pl.BlockSpec((rows_blk, 1), lambda i: (i, 0)),
        out_shape=jax.ShapeDtypeStruct((SEGS, 1), _F32),
    )(x, w1, b1, w2, b2, w3row, b3)


def kernel(query_tokens, document_tokens, table, bn_gamma, bn_beta, bn_mean, bn_var,
           W1, b1, W2, b2, W3, b3):
    qtok = _gather_index(query_tokens.reshape(-1))
    dtok = _gather_index(document_tokens.reshape(-1))
    table_lin = _repack(table.T).reshape(_RP_ROWS * 4, D)

    x = _make_sc_pool()(qtok, dtok, table_lin)

    inv_std = lax.rsqrt(bn_var + 1e-3)
    scale = bn_gamma * inv_std
    shift = bn_beta - bn_mean * scale
    div = jnp.concatenate(
        [jnp.full((D,), 1.0 / L_Q, _F32), jnp.full((D,), 1.0 / L_D, _F32)]
    )
    w1p = (scale * div)[:, None] * W1
    b1p = (shift @ W1 + b1).reshape(1, 64)

    scores = _mlp(
        x, w1p, b1p, W2, b2.reshape(1, 32), W3.reshape(1, 32), b3.reshape(1, 1)
    )
    return scores.reshape(B, N_DOCS)
